# baseline (speedup 1.0000x reference)
import functools

import jax
import jax.numpy as jnp
from jax import lax
from jax.experimental import pallas as pl
from jax.experimental.pallas import tpu as pltpu
from jax.experimental.pallas import tpu_sc as plsc

D_MODEL = 1024
N_HEADS = 8
D_KEYS = 1024
HALF = D_KEYS // 2
N_KEYS = 256
N_EXPERTS = N_KEYS * N_KEYS
TOP_K = 2
TOKENS = 4096

BT = 256
NW = 32
TPW = TOKENS // NW
NSEL = N_HEADS * TOP_K


def _topk2(s, iota_l):
    big = jnp.int32(1 << 30)
    m1 = jnp.max(s, axis=1)
    i1 = jnp.min(jnp.where(s == m1[:, None], iota_l, big), axis=1)
    neg = jnp.where(iota_l == i1[:, None], jnp.float32(-1e30), s)
    m2 = jnp.max(neg, axis=1)
    i2 = jnp.min(jnp.where(neg == m2[:, None], iota_l, big), axis=1)
    return m1, i1, m2, i2


def _phase_a_body(x_ref, w_ref, b_ref, k_ref, q_ref, idx_ref, wgt_ref):
    x = x_ref[...]
    w = w_ref[...]
    q = lax.dot_general(x, w, (((1,), (1,)), ((), ())),
                        preferred_element_type=jnp.float32)
    q = q + b_ref[0]
    q_ref[...] = q

    iota_l = lax.broadcasted_iota(jnp.int32, (BT, N_KEYS), 1)
    outs = []
    for half in range(2):
        qs = q[:, half * HALF:(half + 1) * HALF].astype(jnp.bfloat16)
        kk = k_ref[half]
        s = lax.dot_general(qs, kk, (((1,), (1,)), ((), ())),
                            preferred_element_type=jnp.float32)
        outs.append(_topk2(s, iota_l))
    (m1a, i1a, m2a, i2a), (m1b, i1b, m2b, i2b) = outs
    s11 = m1a + m1b
    i11 = i1a * N_KEYS + i1b
    s12 = m1a + m2b
    i12 = i1a * N_KEYS + i2b
    s21 = m2a + m1b
    i21 = i2a * N_KEYS + i1b
    pick12 = s12 >= s21
    s2nd = jnp.where(pick12, s12, s21)
    i2nd = jnp.where(pick12, i12, i21)
    e = jnp.exp(s2nd - s11)
    w1 = 1.0 / (1.0 + e)
    w2 = e / (1.0 + e)

    iota2 = lax.broadcasted_iota(jnp.int32, (BT, TOP_K), 1)
    idx_ref[0] = jnp.where(iota2 == 0, i11[:, None], i2nd[:, None])
    wgt_ref[0] = jnp.where(iota2 == 0, w1[:, None], w2[:, None])


def _phase_a(xb, wb, b2, kb, ntok):
    grid = (N_HEADS, ntok // BT)
    return pl.pallas_call(
        _phase_a_body,
        grid=grid,
        in_specs=[
            pl.BlockSpec((BT, D_MODEL), lambda h, i: (i, 0)),
            pl.BlockSpec((D_KEYS, D_MODEL), lambda h, i: (h, 0)),
            pl.BlockSpec((1, 1, D_KEYS), lambda h, i: (h, 0, 0)),
            pl.BlockSpec((2, N_KEYS, HALF), lambda h, i: (h, 0, 0)),
        ],
        out_specs=[
            pl.BlockSpec((BT, D_KEYS), lambda h, i: (i, h)),
            pl.BlockSpec((1, BT, TOP_K), lambda h, i: (h, i, 0)),
            pl.BlockSpec((1, BT, TOP_K), lambda h, i: (h, i, 0)),
        ],
        out_shape=[
            jax.ShapeDtypeStruct((ntok, N_HEADS * D_KEYS), jnp.float32),
            jax.ShapeDtypeStruct((N_HEADS, ntok, TOP_K), jnp.int32),
            jax.ShapeDtypeStruct((N_HEADS, ntok, TOP_K), jnp.float32),
        ],
    )(xb, wb, b2, kb)


@functools.cache
def _build_phase_b(ntok):
    tpw = ntok // NW
    mesh = plsc.VectorSubcoreMesh(core_axis_name="c", subcore_axis_name="s")
    return functools.partial(
        pl.kernel,
        out_type=jax.ShapeDtypeStruct((ntok, D_MODEL), jnp.float32),
        mesh=mesh,
        scratch_types=[
            pltpu.VMEM((tpw * NSEL,), jnp.int32),
            pltpu.VMEM((tpw * NSEL,), jnp.float32),
            pltpu.VMEM((N_HEADS * D_KEYS,), jnp.float32),
            pltpu.VMEM((N_HEADS * D_KEYS,), jnp.float32),
            pltpu.VMEM((NSEL, D_MODEL), jnp.float32),
            pltpu.VMEM((NSEL, D_MODEL), jnp.float32),
            pltpu.VMEM((NSEL, D_MODEL), jnp.float32),
            pltpu.VMEM((NSEL, D_MODEL), jnp.float32),
            pltpu.VMEM((D_MODEL,), jnp.float32),
            pltpu.VMEM((D_MODEL,), jnp.float32),
            pltpu.SemaphoreType.DMA,
            pltpu.SemaphoreType.DMA,
            pltpu.SemaphoreType.DMA,
            pltpu.SemaphoreType.DMA,
            pltpu.SemaphoreType.DMA,
            pltpu.SemaphoreType.DMA,
            pltpu.SemaphoreType.DMA,
            pltpu.SemaphoreType.DMA,
        ],
    )(functools.partial(_phase_b_body, tpw))


def _phase_b_body(tpw, q_hbm, idx_hbm, wgt_hbm, wd_hbm, wu_hbm, out_hbm,
                  idx_v, wgt_v, qvA, qvB, wdrA, wdrB, wurA, wurB, outvA, outvB,
                  semqA, semdA, semuA, semoA, semqB, semdB, semuB, semoB):
    wid = lax.axis_index("s") * 2 + lax.axis_index("c")
    b0 = wid * tpw
    pltpu.sync_copy(idx_hbm.at[pl.ds(b0 * NSEL, tpw * NSEL)], idx_v)
    pltpu.sync_copy(wgt_hbm.at[pl.ds(b0 * NSEL, tpw * NSEL)], wgt_v)
    iota16 = lax.iota(jnp.int32, 16)
    zero16 = jnp.zeros((16,), jnp.float32)
    perms = [iota16 ^ s for s in (8, 4, 2, 1)]
    UNROLL = 8
    CH = D_MODEL // 16

    def _lane_perm(a, p):
        return a.at[p].get(mode="promise_in_bounds")

    def _allsum(a):
        for p in perms:
            a = a + _lane_perm(a, p)
        return a

    def _issue(t, qv, wdr, wur, semq, semd, semu):
        b = b0 + t
        pltpu.async_copy(q_hbm.at[b], qv, semq)
        idxv = idx_v[pl.ds(t * NSEL, NSEL)]
        pltpu.async_copy(wd_hbm.at[idxv], wdr, semd)
        pltpu.async_copy(wu_hbm.at[idxv], wur, semu)

    def _drain(qv, wdr, wur, semq, semd, semu):
        pltpu.make_async_copy(q_hbm.at[b0], qv, semq).wait()
        pltpu.make_async_copy(wd_hbm.at[pl.ds(0, NSEL)], wdr, semd).wait()
        pltpu.make_async_copy(wu_hbm.at[pl.ds(0, NSEL)], wur, semu).wait()

    def _process(t, cur, nxt):
        qv, wdr, wur, semq, semd, semu, outv, semo = cur
        tn = jnp.minimum(t + 1, tpw - 1)
        _issue(tn, *nxt[:6])
        _drain(qv, wdr, wur, semq, semd, semu)
        @pl.when(t >= 2)
        def _():
            pltpu.make_async_copy(outv, out_hbm.at[b0], semo).wait()
        wrow = wgt_v[pl.ds(t * NSEL, NSEL)]
        vals = []
        for h in range(N_HEADS):
            def dot_body(c, acc, h=h):
                a0, a1 = acc
                base = c * (16 * UNROLL)
                for u in range(UNROLL):
                    sl = pl.ds(base + u * 16, 16)
                    qc = qv[pl.ds(h * D_KEYS + base + u * 16, 16)]
                    a0 = a0 + qc * wdr[2 * h, sl]
                    a1 = a1 + qc * wdr[2 * h + 1, sl]
                return a0, a1
            a0, a1 = lax.fori_loop(0, CH // UNROLL, dot_body, (zero16, zero16))
            g0 = _lane_perm(wrow, jnp.full((16,), 2 * h, jnp.int32))
            g1s = _lane_perm(wrow, jnp.full((16,), 2 * h + 1, jnp.int32))
            vals.append(jnp.maximum(_allsum(a0), 0.0) * g0)
            vals.append(jnp.maximum(_allsum(a1), 0.0) * g1s)

        def acc_body(c, _):
            for u in range(4):
                sl = pl.ds(c * 64 + u * 16, 16)
                acc = vals[0] * wur[0, sl]
                for j in range(1, NSEL):
                    acc = acc + vals[j] * wur[j, sl]
                outv[sl] = acc
            return 0
        lax.fori_loop(0, CH // 4, acc_body, 0)
        pltpu.async_copy(outv, out_hbm.at[b0 + t], semo)

    bufA = (qvA, wdrA, wurA, semqA, semdA, semuA, outvA, semoA)
    bufB = (qvB, wdrB, wurB, semqB, semdB, semuB, outvB, semoB)
    _issue(0, *bufA[:6])

    def pair_body(t2, carry):
        _process(2 * t2, bufA, bufB)
        _process(2 * t2 + 1, bufB, bufA)
        return carry

    lax.fori_loop(0, tpw // 2, pair_body, 0)
    _drain(*bufA[:6])
    pltpu.make_async_copy(outvA, out_hbm.at[b0], semoA).wait()
    pltpu.make_async_copy(outvB, out_hbm.at[b0], semoB).wait()


def kernel(queries, W_q, b_q, keys_p, w_down, w_up):
    N, T, D = queries.shape
    xb = queries.reshape(-1, D).astype(jnp.bfloat16)
    wb = W_q.astype(jnp.bfloat16)
    b2 = b_q.reshape(N_HEADS, 1, D_KEYS)
    kb = keys_p.reshape(N_HEADS * 2, N_KEYS, HALF).astype(jnp.bfloat16)

    nchunk = 4
    ct = TOKENS // nchunk
    phase_b = _build_phase_b(ct)
    outs = []
    a_res = [_phase_a(xb[c * ct:(c + 1) * ct], wb, b2, kb, ct)
             for c in range(nchunk)]
    for c in range(nchunk):
        q32, idx_h, wgt_h = a_res[c]
        idx_flat = idx_h.transpose(1, 0, 2).reshape(-1)
        wgt_flat = wgt_h.transpose(1, 0, 2).reshape(-1)
        outs.append(phase_b(q32, idx_flat, wgt_flat, w_down, w_up))
    out = jnp.concatenate(outs, axis=0)
    return out.reshape(N, T, D)


# speedup vs baseline: 1.0093x; 1.0093x over previous
import functools

import jax
import jax.numpy as jnp
from jax import lax
from jax.experimental import pallas as pl
from jax.experimental.pallas import tpu as pltpu
from jax.experimental.pallas import tpu_sc as plsc

D_MODEL = 1024
N_HEADS = 8
D_KEYS = 1024
HALF = D_KEYS // 2
N_KEYS = 256
N_EXPERTS = N_KEYS * N_KEYS
TOP_K = 2
TOKENS = 4096

BT = 256
NW = 32
TPW = TOKENS // NW
NSEL = N_HEADS * TOP_K


def _topk2(s, iota_l):
    big = jnp.int32(1 << 30)
    m1 = jnp.max(s, axis=1)
    i1 = jnp.min(jnp.where(s == m1[:, None], iota_l, big), axis=1)
    neg = jnp.where(iota_l == i1[:, None], jnp.float32(-1e30), s)
    m2 = jnp.max(neg, axis=1)
    i2 = jnp.min(jnp.where(neg == m2[:, None], iota_l, big), axis=1)
    return m1, i1, m2, i2


def _phase_a_body(x_ref, w_ref, b_ref, k_ref, q_ref, idx_ref, wgt_ref):
    x = x_ref[...]
    w = w_ref[...]
    q = lax.dot_general(x, w, (((1,), (1,)), ((), ())),
                        preferred_element_type=jnp.float32)
    q = q + b_ref[0]
    q_ref[...] = q

    iota_l = lax.broadcasted_iota(jnp.int32, (BT, N_KEYS), 1)
    outs = []
    for half in range(2):
        qs = q[:, half * HALF:(half + 1) * HALF].astype(jnp.bfloat16)
        kk = k_ref[half]
        s = lax.dot_general(qs, kk, (((1,), (1,)), ((), ())),
                            preferred_element_type=jnp.float32)
        outs.append(_topk2(s, iota_l))
    (m1a, i1a, m2a, i2a), (m1b, i1b, m2b, i2b) = outs
    s11 = m1a + m1b
    i11 = i1a * N_KEYS + i1b
    s12 = m1a + m2b
    i12 = i1a * N_KEYS + i2b
    s21 = m2a + m1b
    i21 = i2a * N_KEYS + i1b
    pick12 = s12 >= s21
    s2nd = jnp.where(pick12, s12, s21)
    i2nd = jnp.where(pick12, i12, i21)
    e = jnp.exp(s2nd - s11)
    w1 = 1.0 / (1.0 + e)
    w2 = e / (1.0 + e)

    iota2 = lax.broadcasted_iota(jnp.int32, (BT, TOP_K), 1)
    idx_ref[0] = jnp.where(iota2 == 0, i11[:, None], i2nd[:, None])
    wgt_ref[0] = jnp.where(iota2 == 0, w1[:, None], w2[:, None])


def _phase_a(xb, wb, b2, kb, ntok):
    grid = (N_HEADS, ntok // BT)
    return pl.pallas_call(
        _phase_a_body,
        grid=grid,
        in_specs=[
            pl.BlockSpec((BT, D_MODEL), lambda h, i: (i, 0)),
            pl.BlockSpec((D_KEYS, D_MODEL), lambda h, i: (h, 0)),
            pl.BlockSpec((1, 1, D_KEYS), lambda h, i: (h, 0, 0)),
            pl.BlockSpec((2, N_KEYS, HALF), lambda h, i: (h, 0, 0)),
        ],
        out_specs=[
            pl.BlockSpec((BT, D_KEYS), lambda h, i: (i, h)),
            pl.BlockSpec((1, BT, TOP_K), lambda h, i: (h, i, 0)),
            pl.BlockSpec((1, BT, TOP_K), lambda h, i: (h, i, 0)),
        ],
        out_shape=[
            jax.ShapeDtypeStruct((ntok, N_HEADS * D_KEYS), jnp.float32),
            jax.ShapeDtypeStruct((N_HEADS, ntok, TOP_K), jnp.int32),
            jax.ShapeDtypeStruct((N_HEADS, ntok, TOP_K), jnp.float32),
        ],
    )(xb, wb, b2, kb)


@functools.cache
def _build_phase_b(ntok):
    tpw = ntok // NW
    mesh = plsc.VectorSubcoreMesh(core_axis_name="c", subcore_axis_name="s")
    return functools.partial(
        pl.kernel,
        out_type=jax.ShapeDtypeStruct((ntok, D_MODEL), jnp.float32),
        mesh=mesh,
        scratch_types=[
            pltpu.VMEM((tpw * NSEL,), jnp.int32),
            pltpu.VMEM((tpw * NSEL,), jnp.float32),
            pltpu.VMEM((N_HEADS * D_KEYS,), jnp.float32),
            pltpu.VMEM((N_HEADS * D_KEYS,), jnp.float32),
            pltpu.VMEM((NSEL, D_MODEL), jnp.float32),
            pltpu.VMEM((NSEL, D_MODEL), jnp.float32),
            pltpu.VMEM((NSEL, D_MODEL), jnp.float32),
            pltpu.VMEM((NSEL, D_MODEL), jnp.float32),
            pltpu.VMEM((D_MODEL,), jnp.float32),
            pltpu.VMEM((D_MODEL,), jnp.float32),
            pltpu.SemaphoreType.DMA,
            pltpu.SemaphoreType.DMA,
            pltpu.SemaphoreType.DMA,
            pltpu.SemaphoreType.DMA,
            pltpu.SemaphoreType.DMA,
            pltpu.SemaphoreType.DMA,
            pltpu.SemaphoreType.DMA,
            pltpu.SemaphoreType.DMA,
        ],
    )(functools.partial(_phase_b_body, tpw))


def _phase_b_body(tpw, q_hbm, idx_hbm, wgt_hbm, wd_hbm, wu_hbm, out_hbm,
                  idx_v, wgt_v, qvA, qvB, wdrA, wdrB, wurA, wurB, outvA, outvB,
                  semqA, semdA, semuA, semoA, semqB, semdB, semuB, semoB):
    wid = lax.axis_index("s") * 2 + lax.axis_index("c")
    b0 = wid * tpw
    pltpu.sync_copy(idx_hbm.at[pl.ds(b0 * NSEL, tpw * NSEL)], idx_v)
    pltpu.sync_copy(wgt_hbm.at[pl.ds(b0 * NSEL, tpw * NSEL)], wgt_v)
    iota16 = lax.iota(jnp.int32, 16)
    zero16 = jnp.zeros((16,), jnp.float32)
    perms = [iota16 ^ s for s in (8, 4, 2, 1)]
    UNROLL = 8
    CH = D_MODEL // 16

    def _lane_perm(a, p):
        return a.at[p].get(mode="promise_in_bounds")

    def _allsum(a):
        for p in perms:
            a = a + _lane_perm(a, p)
        return a

    def _issue(t, qv, wdr, wur, semq, semd, semu):
        b = b0 + t
        pltpu.async_copy(q_hbm.at[b], qv, semq)
        idxv = idx_v[pl.ds(t * NSEL, NSEL)]
        pltpu.async_copy(wd_hbm.at[idxv], wdr, semd)
        pltpu.async_copy(wu_hbm.at[idxv], wur, semu)

    def _drain(qv, wdr, wur, semq, semd, semu):
        pltpu.make_async_copy(q_hbm.at[b0], qv, semq).wait()
        pltpu.make_async_copy(wd_hbm.at[pl.ds(0, NSEL)], wdr, semd).wait()
        pltpu.make_async_copy(wu_hbm.at[pl.ds(0, NSEL)], wur, semu).wait()

    def _process(t, cur, nxt):
        qv, wdr, wur, semq, semd, semu, outv, semo = cur
        tn = jnp.minimum(t + 1, tpw - 1)
        _issue(tn, *nxt[:6])
        _drain(qv, wdr, wur, semq, semd, semu)
        @pl.when(t >= 2)
        def _():
            pltpu.make_async_copy(outv, out_hbm.at[b0], semo).wait()
        wrow = wgt_v[pl.ds(t * NSEL, NSEL)]
        vals = []
        for h in range(N_HEADS):
            def dot_body(c, acc, h=h):
                a0, a1 = acc
                base = c * (16 * UNROLL)
                for u in range(UNROLL):
                    sl = pl.ds(base + u * 16, 16)
                    qc = qv[pl.ds(h * D_KEYS + base + u * 16, 16)]
                    a0 = a0 + qc * wdr[2 * h, sl]
                    a1 = a1 + qc * wdr[2 * h + 1, sl]
                return a0, a1
            a0, a1 = lax.fori_loop(0, CH // UNROLL, dot_body, (zero16, zero16))
            g0 = _lane_perm(wrow, jnp.full((16,), 2 * h, jnp.int32))
            g1s = _lane_perm(wrow, jnp.full((16,), 2 * h + 1, jnp.int32))
            vals.append(jnp.maximum(_allsum(a0), 0.0) * g0)
            vals.append(jnp.maximum(_allsum(a1), 0.0) * g1s)

        def acc_body(c, _):
            for u in range(2):
                sl = pl.ds(c * 32 + u * 16, 16)
                acc = vals[0] * wur[0, sl]
                for j in range(1, NSEL):
                    acc = acc + vals[j] * wur[j, sl]
                outv[sl] = acc
            return 0
        lax.fori_loop(0, CH // 2, acc_body, 0)
        pltpu.async_copy(outv, out_hbm.at[b0 + t], semo)

    bufA = (qvA, wdrA, wurA, semqA, semdA, semuA, outvA, semoA)
    bufB = (qvB, wdrB, wurB, semqB, semdB, semuB, outvB, semoB)
    _issue(0, *bufA[:6])

    def pair_body(t2, carry):
        _process(2 * t2, bufA, bufB)
        _process(2 * t2 + 1, bufB, bufA)
        return carry

    lax.fori_loop(0, tpw // 2, pair_body, 0)
    _drain(*bufA[:6])
    pltpu.make_async_copy(outvA, out_hbm.at[b0], semoA).wait()
    pltpu.make_async_copy(outvB, out_hbm.at[b0], semoB).wait()


def kernel(queries, W_q, b_q, keys_p, w_down, w_up):
    N, T, D = queries.shape
    xb = queries.reshape(-1, D).astype(jnp.bfloat16)
    wb = W_q.astype(jnp.bfloat16)
    b2 = b_q.reshape(N_HEADS, 1, D_KEYS)
    kb = keys_p.reshape(N_HEADS * 2, N_KEYS, HALF).astype(jnp.bfloat16)

    sizes = (512, 1024, 1280, 1280)
    offs = [0]
    for s in sizes:
        offs.append(offs[-1] + s)
    outs = []
    a_res = [_phase_a(xb[offs[c]:offs[c + 1]], wb, b2, kb, sizes[c])
             for c in range(len(sizes))]
    for c in range(len(sizes)):
        q32, idx_h, wgt_h = a_res[c]
        idx_flat = idx_h.transpose(1, 0, 2).reshape(-1)
        wgt_flat = wgt_h.transpose(1, 0, 2).reshape(-1)
        outs.append(_build_phase_b(sizes[c])(q32, idx_flat, wgt_flat, w_down, w_up))
    out = jnp.concatenate(outs, axis=0)
    return out.reshape(N, T, D)


# speedup vs baseline: 1.0355x; 1.0260x over previous
import functools

import jax
import jax.numpy as jnp
from jax import lax
from jax.experimental import pallas as pl
from jax.experimental.pallas import tpu as pltpu
from jax.experimental.pallas import tpu_sc as plsc

D_MODEL = 1024
N_HEADS = 8
D_KEYS = 1024
HALF = D_KEYS // 2
N_KEYS = 256
N_EXPERTS = N_KEYS * N_KEYS
TOP_K = 2
TOKENS = 4096

BT = 256
NW = 32
TPW = TOKENS // NW
NSEL = N_HEADS * TOP_K


def _topk2(s, iota_l):
    big = jnp.int32(1 << 30)
    m1 = jnp.max(s, axis=1)
    i1 = jnp.min(jnp.where(s == m1[:, None], iota_l, big), axis=1)
    neg = jnp.where(iota_l == i1[:, None], jnp.float32(-1e30), s)
    m2 = jnp.max(neg, axis=1)
    i2 = jnp.min(jnp.where(neg == m2[:, None], iota_l, big), axis=1)
    return m1, i1, m2, i2


def _phase_a_body(x_ref, w_ref, b_ref, k_ref, q_ref, idx_ref, wgt_ref):
    x = x_ref[...]
    w = w_ref[...]
    q = lax.dot_general(x, w, (((1,), (1,)), ((), ())),
                        preferred_element_type=jnp.float32)
    q = q + b_ref[0]
    q_ref[...] = q

    iota_l = lax.broadcasted_iota(jnp.int32, (BT, N_KEYS), 1)
    outs = []
    for half in range(2):
        qs = q[:, half * HALF:(half + 1) * HALF].astype(jnp.bfloat16)
        kk = k_ref[half]
        s = lax.dot_general(qs, kk, (((1,), (1,)), ((), ())),
                            preferred_element_type=jnp.float32)
        outs.append(_topk2(s, iota_l))
    (m1a, i1a, m2a, i2a), (m1b, i1b, m2b, i2b) = outs
    s11 = m1a + m1b
    i11 = i1a * N_KEYS + i1b
    s12 = m1a + m2b
    i12 = i1a * N_KEYS + i2b
    s21 = m2a + m1b
    i21 = i2a * N_KEYS + i1b
    pick12 = s12 >= s21
    s2nd = jnp.where(pick12, s12, s21)
    i2nd = jnp.where(pick12, i12, i21)
    e = jnp.exp(s2nd - s11)
    w1 = 1.0 / (1.0 + e)
    w2 = e / (1.0 + e)

    iota2 = lax.broadcasted_iota(jnp.int32, (BT, TOP_K), 1)
    idx_ref[0] = jnp.where(iota2 == 0, i11[:, None], i2nd[:, None])
    wgt_ref[0] = jnp.where(iota2 == 0, w1[:, None], w2[:, None])


def _phase_a(xb, wb, b2, kb, ntok):
    grid = (N_HEADS, ntok // BT)
    return pl.pallas_call(
        _phase_a_body,
        grid=grid,
        in_specs=[
            pl.BlockSpec((BT, D_MODEL), lambda h, i: (i, 0)),
            pl.BlockSpec((D_KEYS, D_MODEL), lambda h, i: (h, 0)),
            pl.BlockSpec((1, 1, D_KEYS), lambda h, i: (h, 0, 0)),
            pl.BlockSpec((2, N_KEYS, HALF), lambda h, i: (h, 0, 0)),
        ],
        out_specs=[
            pl.BlockSpec((BT, D_KEYS), lambda h, i: (i, h)),
            pl.BlockSpec((1, BT, TOP_K), lambda h, i: (h, i, 0)),
            pl.BlockSpec((1, BT, TOP_K), lambda h, i: (h, i, 0)),
        ],
        out_shape=[
            jax.ShapeDtypeStruct((ntok, N_HEADS * D_KEYS), jnp.float32),
            jax.ShapeDtypeStruct((N_HEADS, ntok, TOP_K), jnp.int32),
            jax.ShapeDtypeStruct((N_HEADS, ntok, TOP_K), jnp.float32),
        ],
    )(xb, wb, b2, kb)


@functools.cache
def _build_phase_b(ntok):
    tpw = ntok // NW
    mesh = plsc.VectorSubcoreMesh(core_axis_name="c", subcore_axis_name="s")
    return functools.partial(
        pl.kernel,
        out_type=jax.ShapeDtypeStruct((ntok, D_MODEL), jnp.float32),
        mesh=mesh,
        scratch_types=[
            pltpu.VMEM((tpw * NSEL,), jnp.int32),
            pltpu.VMEM((tpw * NSEL,), jnp.float32),
            pltpu.VMEM((N_HEADS * D_KEYS,), jnp.float32),
            pltpu.VMEM((N_HEADS * D_KEYS,), jnp.float32),
            pltpu.VMEM((NSEL, D_MODEL), jnp.float32),
            pltpu.VMEM((NSEL, D_MODEL), jnp.float32),
            pltpu.VMEM((NSEL, D_MODEL), jnp.float32),
            pltpu.VMEM((NSEL, D_MODEL), jnp.float32),
            pltpu.VMEM((D_MODEL,), jnp.float32),
            pltpu.VMEM((D_MODEL,), jnp.float32),
            pltpu.SemaphoreType.DMA,
            pltpu.SemaphoreType.DMA,
            pltpu.SemaphoreType.DMA,
            pltpu.SemaphoreType.DMA,
            pltpu.SemaphoreType.DMA,
            pltpu.SemaphoreType.DMA,
            pltpu.SemaphoreType.DMA,
            pltpu.SemaphoreType.DMA,
        ],
    )(functools.partial(_phase_b_body, tpw))


def _phase_b_body(tpw, q_hbm, idx_hbm, wgt_hbm, wd_hbm, wu_hbm, out_hbm,
                  idx_v, wgt_v, qvA, qvB, wdrA, wdrB, wurA, wurB, outvA, outvB,
                  semqA, semdA, semuA, semoA, semqB, semdB, semuB, semoB):
    wid = lax.axis_index("s") * 2 + lax.axis_index("c")
    b0 = wid * tpw
    pltpu.sync_copy(idx_hbm.at[pl.ds(b0 * NSEL, tpw * NSEL)], idx_v)
    pltpu.sync_copy(wgt_hbm.at[pl.ds(b0 * NSEL, tpw * NSEL)], wgt_v)
    iota16 = lax.iota(jnp.int32, 16)
    zero16 = jnp.zeros((16,), jnp.float32)
    perms = [iota16 ^ s for s in (8, 4, 2, 1)]
    UNROLL = 8
    CH = D_MODEL // 16

    def _lane_perm(a, p):
        return a.at[p].get(mode="promise_in_bounds")

    def _allsum(a):
        for p in perms:
            a = a + _lane_perm(a, p)
        return a

    def _issue(t, qv, wdr, wur, semq, semd, semu):
        b = b0 + t
        pltpu.async_copy(q_hbm.at[b], qv, semq)
        idxv = idx_v[pl.ds(t * NSEL, NSEL)]
        pltpu.async_copy(wd_hbm.at[idxv], wdr, semd)
        pltpu.async_copy(wu_hbm.at[idxv], wur, semu)

    def _drain(qv, wdr, wur, semq, semd, semu):
        pltpu.make_async_copy(q_hbm.at[b0], qv, semq).wait()
        pltpu.make_async_copy(wd_hbm.at[pl.ds(0, NSEL)], wdr, semd).wait()
        pltpu.make_async_copy(wu_hbm.at[pl.ds(0, NSEL)], wur, semu).wait()

    def _process(t, cur, nxt):
        qv, wdr, wur, semq, semd, semu, outv, semo = cur
        tn = jnp.minimum(t + 1, tpw - 1)
        _issue(tn, *nxt[:6])
        _drain(qv, wdr, wur, semq, semd, semu)
        @pl.when(t >= 2)
        def _():
            pltpu.make_async_copy(outv, out_hbm.at[b0], semo).wait()
        wrow = wgt_v[pl.ds(t * NSEL, NSEL)]
        vals = []
        for h in range(N_HEADS):
            def dot_body(c, acc, h=h):
                a0, a1 = acc
                base = c * (16 * UNROLL)
                for u in range(UNROLL):
                    sl = pl.ds(base + u * 16, 16)
                    qc = qv[pl.ds(h * D_KEYS + base + u * 16, 16)]
                    a0 = a0 + qc * wdr[2 * h, sl]
                    a1 = a1 + qc * wdr[2 * h + 1, sl]
                return a0, a1
            a0, a1 = lax.fori_loop(0, CH // UNROLL, dot_body, (zero16, zero16))
            g0 = _lane_perm(wrow, jnp.full((16,), 2 * h, jnp.int32))
            g1s = _lane_perm(wrow, jnp.full((16,), 2 * h + 1, jnp.int32))
            vals.append(jnp.maximum(_allsum(a0), 0.0) * g0)
            vals.append(jnp.maximum(_allsum(a1), 0.0) * g1s)

        def acc_body(c, _):
            for u in range(2):
                sl = pl.ds(c * 32 + u * 16, 16)
                acc = vals[0] * wur[0, sl]
                for j in range(1, NSEL):
                    acc = acc + vals[j] * wur[j, sl]
                outv[sl] = acc
            return 0
        lax.fori_loop(0, CH // 2, acc_body, 0)
        pltpu.async_copy(outv, out_hbm.at[b0 + t], semo)

    bufA = (qvA, wdrA, wurA, semqA, semdA, semuA, outvA, semoA)
    bufB = (qvB, wdrB, wurB, semqB, semdB, semuB, outvB, semoB)
    _issue(0, *bufA[:6])

    def pair_body(t2, carry):
        _process(2 * t2, bufA, bufB)
        _process(2 * t2 + 1, bufB, bufA)
        return carry

    lax.fori_loop(0, tpw // 2, pair_body, 0)
    _drain(*bufA[:6])
    pltpu.make_async_copy(outvA, out_hbm.at[b0], semoA).wait()
    pltpu.make_async_copy(outvB, out_hbm.at[b0], semoB).wait()


def kernel(queries, W_q, b_q, keys_p, w_down, w_up):
    N, T, D = queries.shape
    xb = queries.reshape(-1, D).astype(jnp.bfloat16)
    wb = W_q.astype(jnp.bfloat16)
    b2 = b_q.reshape(N_HEADS, 1, D_KEYS)
    kb = keys_p.reshape(N_HEADS * 2, N_KEYS, HALF).astype(jnp.bfloat16)

    nchunk = 4
    ct = TOKENS // nchunk
    phase_b = _build_phase_b(ct)
    outs = []
    a_res = [_phase_a(xb[c * ct:(c + 1) * ct], wb, b2, kb, ct)
             for c in range(nchunk)]
    for c in range(nchunk):
        q32, idx_h, wgt_h = a_res[c]
        idx_flat = idx_h.transpose(1, 0, 2).reshape(-1)
        wgt_flat = wgt_h.transpose(1, 0, 2).reshape(-1)
        outs.append(phase_b(q32, idx_flat, wgt_flat, w_down, w_up))
    out = jnp.concatenate(outs, axis=0)
    return out.reshape(N, T, D)
